# Initial kernel scaffold; baseline (speedup 1.0000x reference)
import jax
import jax.numpy as jnp
from jax.experimental import pallas as pl


def kernel(xs_pad, ilens, ys_pad, W1, b1, W2, b2):
    raise NotImplementedError("write your pallas kernel here")


# speedup vs baseline: 7.2973x; 7.2973x over previous
import functools

import jax
import jax.numpy as jnp
from jax import lax
from jax.experimental import pallas as pl
from jax.experimental.pallas import tpu as pltpu

B, T, IDIM = 8, 2048, 80
HDIM, CDIM, TNUM = 160, 16, 10
NLOOP = HDIM // CDIM
K2_MEAN = sum((k + 1) ** 2 for k in range(NLOOP)) / NLOOP
K1_MEAN = sum((k + 1) for k in range(NLOOP)) / NLOOP


def _loss_kernel(ilens_ref, x_ref, w1_ref, b1_ref, w2_ref, b2_ref, out_ref):
    b = pl.program_id(0)
    x = x_ref[0]

    h = jnp.tanh(
        lax.dot_general(x, w1_ref[...], (((1,), (0,)), ((), ())),
                        precision=lax.Precision.HIGHEST,
                        preferred_element_type=jnp.float32)
        + b1_ref[...]
    )
    e = (
        lax.dot_general(h, w2_ref[...], (((1,), (0,)), ((), ())),
                        precision=lax.Precision.HIGHEST,
                        preferred_element_type=jnp.float32)
        + b2_ref[...]
    )

    msum = jnp.concatenate([x[1:], x[:1]], axis=0)
    for i in range(2, TNUM + 1):
        msum = msum + jnp.concatenate([x[i:], x[:i]], axis=0)
    m = msum * (1.0 / TNUM)

    t_idx = lax.broadcasted_iota(jnp.int32, (T, 1), 0)
    v = (t_idx < (ilens_ref[b] - TNUM)).astype(jnp.float32)

    ev = e * v
    mv = m * v
    a_part = jnp.sum(ev * e)
    b_part = jnp.sum(ev * m)
    c_part = jnp.sum(mv * m)
    part = K2_MEAN * a_part - 2.0 * K1_MEAN * b_part + c_part

    @pl.when(b == 0)
    def _():
        out_ref[0, 0] = 0.0

    out_ref[0, 0] += part


@jax.jit
def _run(xs_pad, ilens, W1, b1, W2, b2):
    grid_spec = pltpu.PrefetchScalarGridSpec(
        num_scalar_prefetch=1,
        grid=(B,),
        in_specs=[
            pl.BlockSpec((1, T, IDIM), lambda b, ilens: (b, 0, 0)),
            pl.BlockSpec((IDIM, HDIM), lambda b, ilens: (0, 0)),
            pl.BlockSpec((1, HDIM), lambda b, ilens: (0, 0)),
            pl.BlockSpec((HDIM, IDIM), lambda b, ilens: (0, 0)),
            pl.BlockSpec((1, IDIM), lambda b, ilens: (0, 0)),
        ],
        out_specs=pl.BlockSpec(memory_space=pltpu.SMEM),
    )
    out = pl.pallas_call(
        _loss_kernel,
        grid_spec=grid_spec,
        out_shape=jax.ShapeDtypeStruct((1, 1), jnp.float32),
    )(ilens.astype(jnp.int32), xs_pad,
      W1, b1.reshape(1, HDIM), W2, b2.reshape(1, IDIM))
    return out[0, 0]


def kernel(xs_pad, ilens, ys_pad, W1, b1, W2, b2):
    del ys_pad
    return _run(xs_pad, ilens, W1, b1, W2, b2)


# speedup vs baseline: 18.5676x; 2.5444x over previous
import functools

import jax
import jax.numpy as jnp
from jax import lax
from jax.experimental import pallas as pl
from jax.experimental.pallas import tpu as pltpu

B, T, IDIM = 8, 2048, 80
HDIM, CDIM, TNUM = 160, 16, 10
NLOOP = HDIM // CDIM
K2_MEAN = sum((k + 1) ** 2 for k in range(NLOOP)) / NLOOP
K1_MEAN = sum((k + 1) for k in range(NLOOP)) / NLOOP


def _loss_kernel(ilens_ref, x_ref, w1_ref, b1_ref, w2_ref, b2_ref, out_ref):
    b = pl.program_id(0)
    x = x_ref[0]

    h = jnp.tanh(
        lax.dot_general(x, w1_ref[...], (((1,), (0,)), ((), ())),
                        preferred_element_type=jnp.float32)
        + b1_ref[...]
    )
    e = (
        lax.dot_general(h, w2_ref[...], (((1,), (0,)), ((), ())),
                        preferred_element_type=jnp.float32)
        + b2_ref[...]
    )

    msum = jnp.concatenate([x[1:], x[:1]], axis=0)
    for i in range(2, TNUM + 1):
        msum = msum + jnp.concatenate([x[i:], x[:i]], axis=0)
    m = msum * (1.0 / TNUM)

    t_idx = lax.broadcasted_iota(jnp.int32, (T, 1), 0)
    v = (t_idx < (ilens_ref[b] - TNUM)).astype(jnp.float32)

    ev = e * v
    mv = m * v
    a_part = jnp.sum(ev * e)
    b_part = jnp.sum(ev * m)
    c_part = jnp.sum(mv * m)
    part = K2_MEAN * a_part - 2.0 * K1_MEAN * b_part + c_part

    @pl.when(b == 0)
    def _():
        out_ref[0, 0] = 0.0

    out_ref[0, 0] += part


@jax.jit
def _run(xs_pad, ilens, W1, b1, W2, b2):
    grid_spec = pltpu.PrefetchScalarGridSpec(
        num_scalar_prefetch=1,
        grid=(B,),
        in_specs=[
            pl.BlockSpec((1, T, IDIM), lambda b, ilens: (b, 0, 0)),
            pl.BlockSpec((IDIM, HDIM), lambda b, ilens: (0, 0)),
            pl.BlockSpec((1, HDIM), lambda b, ilens: (0, 0)),
            pl.BlockSpec((HDIM, IDIM), lambda b, ilens: (0, 0)),
            pl.BlockSpec((1, IDIM), lambda b, ilens: (0, 0)),
        ],
        out_specs=pl.BlockSpec(memory_space=pltpu.SMEM),
    )
    out = pl.pallas_call(
        _loss_kernel,
        grid_spec=grid_spec,
        out_shape=jax.ShapeDtypeStruct((1, 1), jnp.float32),
    )(ilens.astype(jnp.int32), xs_pad,
      W1, b1.reshape(1, HDIM), W2, b2.reshape(1, IDIM))
    return out[0, 0]


def kernel(xs_pad, ilens, ys_pad, W1, b1, W2, b2):
    del ys_pad
    return _run(xs_pad, ilens, W1, b1, W2, b2)


# speedup vs baseline: 22.6858x; 1.2218x over previous
import functools

import jax
import jax.numpy as jnp
from jax import lax
from jax.experimental import pallas as pl
from jax.experimental.pallas import tpu as pltpu

B, T, IDIM = 8, 2048, 80
HDIM, CDIM, TNUM = 160, 16, 10
NLOOP = HDIM // CDIM
K2_MEAN = sum((k + 1) ** 2 for k in range(NLOOP)) / NLOOP
K1_MEAN = sum((k + 1) for k in range(NLOOP)) / NLOOP


def _loss_kernel(ilens_ref, x_ref, w1_ref, b1_ref, w2_ref, b2_ref, out_ref):
    b = pl.program_id(0)
    x = x_ref[0]

    h = jnp.tanh(
        lax.dot_general(x, w1_ref[...], (((1,), (0,)), ((), ())),
                        preferred_element_type=jnp.float32)
        + b1_ref[...]
    )
    e = (
        lax.dot_general(h, w2_ref[...], (((1,), (0,)), ((), ())),
                        preferred_element_type=jnp.float32)
        + b2_ref[...]
    )

    def s(a, i):
        return jnp.concatenate([a[i:], a[:i]], axis=0)

    u = s(x, 1) + s(x, 2)
    w = u + s(u, 2)
    w = w + s(w, 4)
    m = (w + s(u, 8)) * (1.0 / TNUM)

    t_idx = lax.broadcasted_iota(jnp.int32, (T, 1), 0)
    vmask = (t_idx < (ilens_ref[b] - TNUM)).astype(jnp.float32)

    q = e * vmask
    p = m * vmask
    a_part = jnp.sum(q * q)
    b_part = jnp.sum(q * p)
    c_part = jnp.sum(p * p)
    part = K2_MEAN * a_part - 2.0 * K1_MEAN * b_part + c_part

    @pl.when(b == 0)
    def _():
        out_ref[0, 0] = 0.0

    out_ref[0, 0] += part


@jax.jit
def _run(xs_pad, ilens, W1, b1, W2, b2):
    grid_spec = pltpu.PrefetchScalarGridSpec(
        num_scalar_prefetch=1,
        grid=(B,),
        in_specs=[
            pl.BlockSpec((1, T, IDIM), lambda b, ilens: (b, 0, 0)),
            pl.BlockSpec((IDIM, HDIM), lambda b, ilens: (0, 0)),
            pl.BlockSpec((1, HDIM), lambda b, ilens: (0, 0)),
            pl.BlockSpec((HDIM, IDIM), lambda b, ilens: (0, 0)),
            pl.BlockSpec((1, IDIM), lambda b, ilens: (0, 0)),
        ],
        out_specs=pl.BlockSpec(memory_space=pltpu.SMEM),
    )
    out = pl.pallas_call(
        _loss_kernel,
        grid_spec=grid_spec,
        out_shape=jax.ShapeDtypeStruct((1, 1), jnp.float32),
    )(ilens.astype(jnp.int32), xs_pad,
      W1, b1.reshape(1, HDIM), W2, b2.reshape(1, IDIM))
    return out[0, 0]


def kernel(xs_pad, ilens, ys_pad, W1, b1, W2, b2):
    del ys_pad
    return _run(xs_pad, ilens, W1, b1, W2, b2)
